# baseline (speedup 1.0000x reference)
import functools

import jax
import jax.numpy as jnp
from jax import lax
from jax.experimental import pallas as pl
from jax.experimental.pallas import tpu as pltpu
from jax.experimental.pallas import tpu_sc as plsc

N = 320000
D = 128
R = 20000

NTILES = 32
NPART = 64
RPT = 320
RPAD = RPT * NPART
ROWS = 324
DUMP = 320
C = 128

_BL = 2560


def _mm_body(x_ref, w_ref, o_ref):
    h = jax.lax.dot_general(x_ref[...], w_ref[...],
                            (((1,), (1,)), ((), ())),
                            preferred_element_type=jnp.float32)
    o_ref[...] = jnp.maximum(h, 0.0).astype(jnp.bfloat16)


def _matmul_relu(x, w):
    return pl.pallas_call(
        _mm_body,
        grid=(N // _BL,),
        in_specs=[pl.BlockSpec((_BL, D), lambda i: (i, 0)),
                  pl.BlockSpec((D, D), lambda i: (0, 0))],
        out_specs=pl.BlockSpec((_BL, D), lambda i: (i, 0)),
        out_shape=jax.ShapeDtypeStruct((N, D), jnp.bfloat16),
    )(x, w)


def _sc_body(hm_hbm, uri_hbm, bounds_hbm,
             out_hbm, cnt_hbm):
    pl.run_scoped(
        functools.partial(_sc_inner, hm_hbm, uri_hbm, bounds_hbm,
                          out_hbm, cnt_hbm),
        pltpu.VMEM((ROWS, D), jnp.float32),
        pltpu.VMEM((ROWS, 16), jnp.float32),
        pltpu.VMEM((C, D // 2), jnp.int32),
        pltpu.VMEM((C, D // 2), jnp.int32),
        pltpu.VMEM((C,), jnp.int32),
        pltpu.VMEM((C,), jnp.int32),
        pltpu.VMEM((80,), jnp.int32),
        pltpu.SemaphoreType.DMA,
        pltpu.SemaphoreType.DMA,
    )


def _sc_inner(hm_hbm, uri_hbm, bounds_hbm,
              out_hbm, cnt_hbm,
              accum, cnta, ch0, ch1, ix0, ix1, bvm, sem0, sem1):
    info = plsc.get_sparse_core_info()
    nc = info.num_cores
    sid = lax.axis_index("s")
    wid = sid * nc + lax.axis_index("c")

    zero16 = jnp.zeros((16,), jnp.float32)
    one16 = jnp.ones((16,), jnp.float32)
    himask = jnp.full((16,), -65536, jnp.int32)

    pltpu.sync_copy(bounds_hbm, bvm)
    bv = bvm[pl.ds(2 * wid, 16)]

    bufs = ((ch0, ix0, sem0), (ch1, ix1, sem1))

    for h in range(2):
        b0 = bv[h]
        b1 = bv[h + 1]
        a0 = (b0 // 8) * 8
        nch = (b1 - a0 + (C - 1)) // C
        nch2 = ((nch + 1) // 2) * 2
        r0 = (2 * wid + h) * RPT

        def _zrow(r, carry):
            for j in range(8):
                accum[r, pl.ds(16 * j, 16)] = zero16
            cnta[r, :] = zero16
            return carry

        lax.fori_loop(0, ROWS, _zrow, 0)

        def _dstart(c, a0=a0):
            return jnp.minimum(a0 + c * C, N - C)

        def _issue(c, buf, a0=a0):
            ch, ix, sem = buf
            ds0 = _dstart(c, a0)
            pltpu.async_copy(hm_hbm.at[pl.ds(ds0, C)], ch, sem)
            pltpu.async_copy(uri_hbm.at[pl.ds(ds0, C)], ix, sem)

        def _drain(buf):
            ch, ix, sem = buf
            pltpu.make_async_copy(hm_hbm.at[pl.ds(0, C)], ch, sem).wait()
            pltpu.make_async_copy(uri_hbm.at[pl.ds(0, C)], ix, sem).wait()

        def _process(c, buf, b0=b0, b1=b1, a0=a0, r0=r0):
            ch, ix, _ = buf
            start = a0 + c * C
            ds0 = _dstart(c, a0)

            def _group(g, carry2):
                pos = ds0 + g * 16 + lax.iota(jnp.int32, 16)
                iv = ix[pl.ds(g * 16, 16)]
                ok = (pos >= b0) & (pos >= start) & (pos < b1) & (iv >= 0)
                lv = jnp.where(ok, iv - r0, DUMP)
                for l in range(16):
                    r = lv[l]
                    a = g * 16 + l
                    plsc.addupdate(cnta.at[r, :], one16)
                    for k in range(4):
                        v = ch[a, pl.ds(16 * k, 16)]
                        fe = lax.bitcast_convert_type(v << 16, jnp.float32)
                        fo = lax.bitcast_convert_type(v & himask, jnp.float32)
                        plsc.addupdate(accum.at[r, pl.ds(32 * k, 16)], fe)
                        plsc.addupdate(accum.at[r, pl.ds(32 * k + 16, 16)], fo)
                return carry2

            lax.fori_loop(0, C // 16, _group, 0)

        _issue(0, bufs[0])

        def _pair(cc, carry):
            _issue(cc + 1, bufs[1])
            _drain(bufs[0])
            _process(cc, bufs[0])
            _issue(cc + 2, bufs[0])
            _drain(bufs[1])
            _process(cc + 1, bufs[1])
            return carry

        lax.fori_loop(0, nch2 // 2, lambda i, c: _pair(2 * i, c), 0)
        _drain(bufs[0])

        def _div(r, carry):
            cv = cnta[r, :]
            scale = 1.0 / jnp.maximum(cv, 1.0)
            for j in range(8):
                sl = pl.ds(16 * j, 16)
                accum[r, sl] = accum[r, sl] * scale
            cnta[r, :] = jnp.where(cv > 0.0, 1.0, 0.0)
            return carry

        lax.fori_loop(0, RPT, _div, 0)

        pltpu.sync_copy(accum.at[pl.ds(0, RPT)], out_hbm.at[pl.ds(r0, RPT)])
        pltpu.sync_copy(cnta.at[pl.ds(0, RPT)], cnt_hbm.at[pl.ds(r0, RPT)])


def _sc_segment_mean(hm32, muri, bounds):
    mesh = plsc.VectorSubcoreMesh(core_axis_name="c", subcore_axis_name="s")
    fn = functools.partial(
        pl.kernel,
        mesh=mesh,
        out_type=[jax.ShapeDtypeStruct((RPAD, D), jnp.float32),
                  jax.ShapeDtypeStruct((RPAD, 16), jnp.float32)],
    )(_sc_body)
    return fn(hm32, muri, bounds)


def _unpack_perm():
    p = [0] * D
    for k in range(4):
        for i in range(16):
            p[32 * k + 2 * i] = 32 * k + i
            p[32 * k + 2 * i + 1] = 32 * k + 16 + i
    return p


@jax.jit
def kernel(x, is_center, unique_residue_index, not_pad_mask, W):
    del is_center
    uri = unique_residue_index.astype(jnp.int32)
    muri = jnp.where(not_pad_mask, uri, -1)

    w_perm = W[jnp.array(_unpack_perm(), jnp.int32), :]
    hm = _matmul_relu(x, w_perm)
    hm32 = lax.bitcast_convert_type(hm.reshape(N, D // 2, 2), jnp.int32)

    edges = jnp.arange(0, RPAD + RPT, RPT, dtype=jnp.int32)
    bounds = jnp.searchsorted(uri, edges).astype(jnp.int32)
    bounds = jnp.concatenate(
        [bounds, jnp.zeros((80 - NPART - 1,), jnp.int32)])

    out, cnt = _sc_segment_mean(hm32, muri, bounds)
    node_emb = out[:R]
    residue_mask = cnt[:R, 0].astype(bool)
    return node_emb, residue_mask


# speedup vs baseline: 3.3211x; 3.3211x over previous
import functools

import jax
import jax.numpy as jnp
from jax import lax
from jax.experimental import pallas as pl
from jax.experimental.pallas import tpu as pltpu
from jax.experimental.pallas import tpu_sc as plsc

N = 320000
D = 128
R = 20000

NTILES = 32
NPART = 64
RPT = 320
RPAD = RPT * NPART
ROWS = 324
DUMP = 320
C = 128

_BL = 2560


def _mm_body(x_ref, w_ref, o_ref):
    h = jax.lax.dot_general(x_ref[...], w_ref[...],
                            (((1,), (1,)), ((), ())),
                            preferred_element_type=jnp.float32)
    o_ref[...] = jnp.maximum(h, 0.0).astype(jnp.bfloat16)


def _matmul_relu(x, w):
    return pl.pallas_call(
        _mm_body,
        grid=(N // _BL,),
        in_specs=[pl.BlockSpec((_BL, D), lambda i: (i, 0)),
                  pl.BlockSpec((D, D), lambda i: (0, 0))],
        out_specs=pl.BlockSpec((_BL, D), lambda i: (i, 0)),
        out_shape=jax.ShapeDtypeStruct((N, D), jnp.bfloat16),
    )(x, w)


def _sc_body(hm_hbm, uri_hbm, bounds_hbm,
             out_hbm, cnt_hbm):
    pl.run_scoped(
        functools.partial(_sc_inner, hm_hbm, uri_hbm, bounds_hbm,
                          out_hbm, cnt_hbm),
        pltpu.VMEM((ROWS, D), jnp.float32),
        pltpu.VMEM((ROWS, 16), jnp.float32),
        pltpu.VMEM((C // 2, D), jnp.int32),
        pltpu.VMEM((C // 2, D), jnp.int32),
        pltpu.VMEM((C,), jnp.int32),
        pltpu.VMEM((C,), jnp.int32),
        pltpu.VMEM((80,), jnp.int32),
        pltpu.SemaphoreType.DMA,
        pltpu.SemaphoreType.DMA,
    )


def _sc_inner(hm_hbm, uri_hbm, bounds_hbm,
              out_hbm, cnt_hbm,
              accum, cnta, ch0, ch1, ix0, ix1, bvm, sem0, sem1):
    info = plsc.get_sparse_core_info()
    nc = info.num_cores
    sid = lax.axis_index("s")
    wid = sid * nc + lax.axis_index("c")

    zero16 = jnp.zeros((16,), jnp.float32)
    one16 = jnp.ones((16,), jnp.float32)
    himask = jnp.full((16,), -65536, jnp.int32)
    hm32_hbm = hm_hbm.bitcast(jnp.int32)
    pltpu.sync_copy(bounds_hbm, bvm)
    bv = bvm[pl.ds(2 * wid, 16)]

    bufs = ((ch0, ix0, sem0), (ch1, ix1, sem1))

    for h in range(2):
        b0 = bv[h]
        b1 = bv[h + 1]
        a0 = (b0 // 16) * 16
        nch = (b1 - a0 + (C - 1)) // C
        nch2 = ((nch + 1) // 2) * 2
        r0 = (2 * wid + h) * RPT

        def _zrow(r, carry):
            for j in range(8):
                accum[r, pl.ds(16 * j, 16)] = zero16
            cnta[r, :] = zero16
            return carry

        lax.fori_loop(0, ROWS, _zrow, 0)

        def _dstart(c, a0=a0):
            return jnp.minimum(a0 + c * C, N - C)

        def _issue(c, buf, a0=a0):
            ch, ix, sem = buf
            ds0 = _dstart(c, a0)
            row0 = pl.multiple_of(ds0 // 2, 8)
            pltpu.async_copy(hm32_hbm.at[pl.ds(row0, C // 2)], ch, sem)
            pltpu.async_copy(uri_hbm.at[pl.ds(ds0, C)], ix, sem)

        def _drain(buf):
            ch, ix, sem = buf
            pltpu.make_async_copy(
                hm32_hbm.at[pl.ds(0, C // 2)], ch, sem).wait()
            pltpu.make_async_copy(uri_hbm.at[pl.ds(0, C)], ix, sem).wait()

        def _process(c, buf, b0=b0, b1=b1, a0=a0, r0=r0):
            ch, ix, _ = buf
            start = a0 + c * C
            ds0 = _dstart(c, a0)

            def _group(g, carry2):
                pos = ds0 + g * 16 + lax.iota(jnp.int32, 16)
                iv = ix[pl.ds(g * 16, 16)]
                ok = (pos >= b0) & (pos >= start) & (pos < b1) & (iv >= 0)
                lv = jnp.where(ok, iv - r0, DUMP)
                for p in range(8):
                    re = lv[2 * p]
                    ro = lv[2 * p + 1]
                    plsc.addupdate(cnta.at[re, :], one16)
                    plsc.addupdate(cnta.at[ro, :], one16)
                    vrow = g * 8 + p
                    for k in range(8):
                        sl = pl.ds(16 * k, 16)
                        v = ch[vrow, sl]
                        fe = lax.bitcast_convert_type(v << 16, jnp.float32)
                        fo = lax.bitcast_convert_type(v & himask, jnp.float32)
                        plsc.addupdate(accum.at[re, sl], fe)
                        plsc.addupdate(accum.at[ro, sl], fo)
                return carry2

            lax.fori_loop(0, C // 16, _group, 0)

        _issue(0, bufs[0])

        def _pair(cc, carry):
            _issue(cc + 1, bufs[1])
            _drain(bufs[0])
            _process(cc, bufs[0])
            _issue(cc + 2, bufs[0])
            _drain(bufs[1])
            _process(cc + 1, bufs[1])
            return carry

        lax.fori_loop(0, nch2 // 2, lambda i, c: _pair(2 * i, c), 0)
        _drain(bufs[0])

        def _div(r, carry):
            cv = cnta[r, :]
            scale = 1.0 / jnp.maximum(cv, 1.0)
            for j in range(8):
                sl = pl.ds(16 * j, 16)
                accum[r, sl] = accum[r, sl] * scale
            cnta[r, :] = jnp.where(cv > 0.0, 1.0, 0.0)
            return carry

        lax.fori_loop(0, RPT, _div, 0)

        pltpu.sync_copy(accum.at[pl.ds(0, RPT)], out_hbm.at[pl.ds(r0, RPT)])
        pltpu.sync_copy(cnta.at[pl.ds(0, RPT)], cnt_hbm.at[pl.ds(r0, RPT)])


def _sc_segment_mean(hm, muri, bounds):
    mesh = plsc.VectorSubcoreMesh(core_axis_name="c", subcore_axis_name="s")
    fn = functools.partial(
        pl.kernel,
        mesh=mesh,
        out_type=[jax.ShapeDtypeStruct((RPAD, D), jnp.float32),
                  jax.ShapeDtypeStruct((RPAD, 16), jnp.float32)],
    )(_sc_body)
    return fn(hm, muri, bounds)


def _unpack_perm():
    p = [0] * D
    for k in range(4):
        for i in range(16):
            p[32 * k + 2 * i] = 32 * k + i
            p[32 * k + 2 * i + 1] = 32 * k + 16 + i
    return p


@jax.jit
def kernel(x, is_center, unique_residue_index, not_pad_mask, W):
    del is_center
    uri = unique_residue_index.astype(jnp.int32)
    muri = jnp.where(not_pad_mask, uri, -1)

    hm = _matmul_relu(x, W)

    edges = jnp.arange(0, RPAD + RPT, RPT, dtype=jnp.int32)
    bounds = jnp.searchsorted(uri, edges).astype(jnp.int32)
    bounds = jnp.concatenate(
        [bounds, jnp.zeros((80 - NPART - 1,), jnp.int32)])

    out, cnt = _sc_segment_mean(hm, muri, bounds)
    node_emb = out[:R]
    residue_mask = cnt[:R, 0].astype(bool)
    return node_emb, residue_mask


# speedup vs baseline: 3.6078x; 1.0863x over previous
import functools

import jax
import jax.numpy as jnp
from jax import lax
from jax.experimental import pallas as pl
from jax.experimental.pallas import tpu as pltpu
from jax.experimental.pallas import tpu_sc as plsc

N = 320000
D = 128
R = 20000

NTILES = 32
NPART = 64
RPT = 320
RPAD = RPT * NPART
ROWS = 324
DUMP = 320
C = 128

_BL = 4000


def _mm_body(x_ref, w_ref, o_ref):
    h = jax.lax.dot_general(x_ref[...].astype(jnp.bfloat16),
                            w_ref[...].astype(jnp.bfloat16),
                            (((1,), (1,)), ((), ())),
                            preferred_element_type=jnp.float32)
    o_ref[...] = jnp.maximum(h, 0.0).astype(jnp.bfloat16)


def _matmul_relu(x, w):
    return pl.pallas_call(
        _mm_body,
        grid=(N // _BL,),
        in_specs=[pl.BlockSpec((_BL, D), lambda i: (i, 0)),
                  pl.BlockSpec((D, D), lambda i: (0, 0))],
        out_specs=pl.BlockSpec((_BL, D), lambda i: (i, 0)),
        out_shape=jax.ShapeDtypeStruct((N, D), jnp.bfloat16),
    )(x, w)


def _sc_body(hm_hbm, uri_hbm, bounds_hbm,
             out_hbm, cnt_hbm):
    pl.run_scoped(
        functools.partial(_sc_inner, hm_hbm, uri_hbm, bounds_hbm,
                          out_hbm, cnt_hbm),
        pltpu.VMEM((ROWS, D), jnp.float32),
        pltpu.VMEM((ROWS, 16), jnp.float32),
        pltpu.VMEM((C // 2, D), jnp.int32),
        pltpu.VMEM((C // 2, D), jnp.int32),
        pltpu.VMEM((C,), jnp.int32),
        pltpu.VMEM((C,), jnp.int32),
        pltpu.VMEM((80,), jnp.int32),
        pltpu.SemaphoreType.DMA,
        pltpu.SemaphoreType.DMA,
    )


def _sc_inner(hm_hbm, uri_hbm, bounds_hbm,
              out_hbm, cnt_hbm,
              accum, cnta, ch0, ch1, ix0, ix1, bvm, sem0, sem1):
    info = plsc.get_sparse_core_info()
    nc = info.num_cores
    sid = lax.axis_index("s")
    wid = sid * nc + lax.axis_index("c")

    zero16 = jnp.zeros((16,), jnp.float32)
    one16 = jnp.ones((16,), jnp.float32)
    himask = jnp.full((16,), -65536, jnp.int32)
    hm32_hbm = hm_hbm.bitcast(jnp.int32)
    pltpu.sync_copy(bounds_hbm, bvm)
    bv = bvm[pl.ds(2 * wid, 16)]

    bufs = ((ch0, ix0, sem0), (ch1, ix1, sem1))

    for h in range(2):
        b0 = bv[h]
        b1 = bv[h + 1]
        a0 = (b0 // 16) * 16
        nch = (b1 - a0 + (C - 1)) // C
        nch2 = ((nch + 1) // 2) * 2
        r0 = (2 * wid + h) * RPT

        def _zrow(r, carry):
            for j in range(8):
                accum[r, pl.ds(16 * j, 16)] = zero16
            cnta[r, :] = zero16
            return carry

        lax.fori_loop(0, ROWS, _zrow, 0)

        def _dstart(c, a0=a0):
            return jnp.minimum(a0 + c * C, N - C)

        def _issue(c, buf, a0=a0):
            ch, ix, sem = buf
            ds0 = _dstart(c, a0)
            row0 = pl.multiple_of(ds0 // 2, 8)
            pltpu.async_copy(hm32_hbm.at[pl.ds(row0, C // 2)], ch, sem)
            pltpu.async_copy(uri_hbm.at[pl.ds(ds0, C)], ix, sem)

        def _drain(buf):
            ch, ix, sem = buf
            pltpu.make_async_copy(
                hm32_hbm.at[pl.ds(0, C // 2)], ch, sem).wait()
            pltpu.make_async_copy(uri_hbm.at[pl.ds(0, C)], ix, sem).wait()

        def _process(c, buf, b0=b0, b1=b1, a0=a0, r0=r0):
            ch, ix, _ = buf
            start = a0 + c * C
            ds0 = _dstart(c, a0)

            def _group(g, carry2):
                pos = ds0 + g * 16 + lax.iota(jnp.int32, 16)
                iv = ix[pl.ds(g * 16, 16)]
                ok = (pos >= b0) & (pos >= start) & (pos < b1) & (iv >= 0)
                lv = jnp.where(ok, iv - r0, DUMP)
                for p in range(8):
                    re = lv[2 * p]
                    ro = lv[2 * p + 1]
                    plsc.addupdate(cnta.at[re, :], one16)
                    plsc.addupdate(cnta.at[ro, :], one16)
                    vrow = g * 8 + p
                    for k in range(8):
                        sl = pl.ds(16 * k, 16)
                        v = ch[vrow, sl]
                        fe = lax.bitcast_convert_type(v << 16, jnp.float32)
                        fo = lax.bitcast_convert_type(v & himask, jnp.float32)
                        plsc.addupdate(accum.at[re, sl], fe)
                        plsc.addupdate(accum.at[ro, sl], fo)
                return carry2

            lax.fori_loop(0, C // 16, _group, 0)

        _issue(0, bufs[0])

        def _pair(cc, carry):
            _issue(cc + 1, bufs[1])
            _drain(bufs[0])
            _process(cc, bufs[0])
            _issue(cc + 2, bufs[0])
            _drain(bufs[1])
            _process(cc + 1, bufs[1])
            return carry

        lax.fori_loop(0, nch2 // 2, lambda i, c: _pair(2 * i, c), 0)
        _drain(bufs[0])

        def _div(r, carry):
            cv = cnta[r, :]
            scale = 1.0 / jnp.maximum(cv, 1.0)
            for j in range(8):
                sl = pl.ds(16 * j, 16)
                accum[r, sl] = accum[r, sl] * scale
            cnta[r, :] = jnp.where(cv > 0.0, 1.0, 0.0)
            return carry

        lax.fori_loop(0, RPT, _div, 0)

        pltpu.sync_copy(accum.at[pl.ds(0, RPT)], out_hbm.at[pl.ds(r0, RPT)])
        pltpu.sync_copy(cnta.at[pl.ds(0, RPT)], cnt_hbm.at[pl.ds(r0, RPT)])


def _sc_segment_mean(hm, muri, bounds):
    mesh = plsc.VectorSubcoreMesh(core_axis_name="c", subcore_axis_name="s")
    fn = functools.partial(
        pl.kernel,
        mesh=mesh,
        out_type=[jax.ShapeDtypeStruct((RPAD, D), jnp.float32),
                  jax.ShapeDtypeStruct((RPAD, 16), jnp.float32)],
    )(_sc_body)
    return fn(hm, muri, bounds)


def _unpack_perm():
    p = [0] * D
    for k in range(4):
        for i in range(16):
            p[32 * k + 2 * i] = 32 * k + i
            p[32 * k + 2 * i + 1] = 32 * k + 16 + i
    return p


@jax.jit
def kernel(x, is_center, unique_residue_index, not_pad_mask, W):
    del is_center
    uri = unique_residue_index.astype(jnp.int32)
    muri = jnp.where(not_pad_mask, uri, -1)

    hm = _matmul_relu(x, W)

    edges = jnp.arange(0, RPAD + RPT, RPT, dtype=jnp.int32)
    bounds = jnp.searchsorted(uri, edges).astype(jnp.int32)
    bounds = jnp.concatenate(
        [bounds, jnp.zeros((80 - NPART - 1,), jnp.int32)])

    out, cnt = _sc_segment_mean(hm, muri, bounds)
    node_emb = out[:R]
    residue_mask = cnt[:R, 0].astype(bool)
    return node_emb, residue_mask


# speedup vs baseline: 3.9048x; 1.0823x over previous
import functools

import jax
import jax.numpy as jnp
from jax import lax
from jax.experimental import pallas as pl
from jax.experimental.pallas import tpu as pltpu
from jax.experimental.pallas import tpu_sc as plsc

N = 320000
D = 128
R = 20000

NTILES = 32
NPART = 64
RPT = 320
RPAD = RPT * NPART
ROWS = 324
DUMP = 320
C = 128

_BL = 8000


def _mm_body(x_ref, w_ref, o_ref):
    h = jax.lax.dot_general(x_ref[...].astype(jnp.bfloat16),
                            w_ref[...].astype(jnp.bfloat16),
                            (((1,), (1,)), ((), ())),
                            preferred_element_type=jnp.float32)
    o_ref[...] = jnp.maximum(h, 0.0).astype(jnp.bfloat16)


def _matmul_relu(x, w):
    return pl.pallas_call(
        _mm_body,
        grid=(N // _BL,),
        in_specs=[pl.BlockSpec((_BL, D), lambda i: (i, 0)),
                  pl.BlockSpec((D, D), lambda i: (0, 0))],
        out_specs=pl.BlockSpec((_BL, D), lambda i: (i, 0)),
        out_shape=jax.ShapeDtypeStruct((N, D), jnp.bfloat16),
    )(x, w)


def _sc_body(hm_hbm, uri_hbm, bounds_hbm,
             out_hbm, cnt_hbm):
    pl.run_scoped(
        functools.partial(_sc_inner, hm_hbm, uri_hbm, bounds_hbm,
                          out_hbm, cnt_hbm),
        pltpu.VMEM((ROWS, D), jnp.float32),
        pltpu.VMEM((ROWS, 16), jnp.float32),
        pltpu.VMEM((C // 2, D), jnp.int32),
        pltpu.VMEM((C // 2, D), jnp.int32),
        pltpu.VMEM((C,), jnp.int32),
        pltpu.VMEM((C,), jnp.int32),
        pltpu.VMEM((80,), jnp.int32),
        pltpu.SemaphoreType.DMA,
        pltpu.SemaphoreType.DMA,
    )


def _sc_inner(hm_hbm, uri_hbm, bounds_hbm,
              out_hbm, cnt_hbm,
              accum, cnta, ch0, ch1, ix0, ix1, bvm, sem0, sem1):
    info = plsc.get_sparse_core_info()
    nc = info.num_cores
    sid = lax.axis_index("s")
    wid = sid * nc + lax.axis_index("c")

    zero16 = jnp.zeros((16,), jnp.float32)
    one16 = jnp.ones((16,), jnp.float32)
    himask = jnp.full((16,), -65536, jnp.int32)
    hm32_hbm = hm_hbm.bitcast(jnp.int32)
    pltpu.sync_copy(bounds_hbm, bvm)
    bv = bvm[pl.ds(2 * wid, 16)]

    bufs = ((ch0, ix0, sem0), (ch1, ix1, sem1))

    for h in range(2):
        b0 = bv[h]
        b1 = bv[h + 1]
        a0 = (b0 // 16) * 16
        nch = (b1 - a0 + (C - 1)) // C
        nch2 = ((nch + 1) // 2) * 2
        r0 = (2 * wid + h) * RPT

        def _zrow(r, carry):
            for j in range(8):
                accum[r, pl.ds(16 * j, 16)] = zero16
            cnta[r, :] = zero16
            return carry

        lax.fori_loop(0, ROWS, _zrow, 0)

        def _dstart(c, a0=a0):
            return jnp.minimum(a0 + c * C, N - C)

        def _issue(c, buf, a0=a0):
            ch, ix, sem = buf
            ds0 = _dstart(c, a0)
            row0 = pl.multiple_of(ds0 // 2, 8)
            pltpu.async_copy(hm32_hbm.at[pl.ds(row0, C // 2)], ch, sem)
            pltpu.async_copy(uri_hbm.at[pl.ds(ds0, C)], ix, sem)

        def _drain(buf):
            ch, ix, sem = buf
            pltpu.make_async_copy(
                hm32_hbm.at[pl.ds(0, C // 2)], ch, sem).wait()
            pltpu.make_async_copy(uri_hbm.at[pl.ds(0, C)], ix, sem).wait()

        def _process(c, buf, b0=b0, b1=b1, a0=a0, r0=r0):
            ch, ix, _ = buf
            start = a0 + c * C
            ds0 = _dstart(c, a0)

            def _group(g, carry2):
                pos = ds0 + g * 16 + lax.iota(jnp.int32, 16)
                iv = ix[pl.ds(g * 16, 16)]
                ok = (pos >= b0) & (pos >= start) & (pos < b1) & (iv >= 0)
                lv = jnp.where(ok, iv - r0, DUMP)
                for p in range(8):
                    re = lv[2 * p]
                    ro = lv[2 * p + 1]
                    plsc.addupdate(cnta.at[re, :], one16)
                    plsc.addupdate(cnta.at[ro, :], one16)
                    vrow = g * 8 + p
                    for k in range(8):
                        sl = pl.ds(16 * k, 16)
                        v = ch[vrow, sl]
                        fe = lax.bitcast_convert_type(v << 16, jnp.float32)
                        fo = lax.bitcast_convert_type(v & himask, jnp.float32)
                        plsc.addupdate(accum.at[re, sl], fe)
                        plsc.addupdate(accum.at[ro, sl], fo)
                return carry2

            lax.fori_loop(0, C // 16, _group, 0)

        _issue(0, bufs[0])

        def _pair(cc, carry):
            _issue(cc + 1, bufs[1])
            _drain(bufs[0])
            _process(cc, bufs[0])
            _issue(cc + 2, bufs[0])
            _drain(bufs[1])
            _process(cc + 1, bufs[1])
            return carry

        lax.fori_loop(0, nch2 // 2, lambda i, c: _pair(2 * i, c), 0)
        _drain(bufs[0])

        def _div(r, carry):
            cv = cnta[r, :]
            scale = 1.0 / jnp.maximum(cv, 1.0)
            for j in range(8):
                sl = pl.ds(16 * j, 16)
                accum[r, sl] = accum[r, sl] * scale
            cnta[r, :] = jnp.where(cv > 0.0, 1.0, 0.0)
            return carry

        lax.fori_loop(0, RPT, _div, 0)

        pltpu.sync_copy(accum.at[pl.ds(0, RPT)], out_hbm.at[pl.ds(r0, RPT)])
        pltpu.sync_copy(cnta.at[pl.ds(0, RPT)], cnt_hbm.at[pl.ds(r0, RPT)])


def _sc_segment_mean(hm, muri, bounds):
    mesh = plsc.VectorSubcoreMesh(core_axis_name="c", subcore_axis_name="s")
    fn = functools.partial(
        pl.kernel,
        mesh=mesh,
        out_type=[jax.ShapeDtypeStruct((RPAD, D), jnp.float32),
                  jax.ShapeDtypeStruct((RPAD, 16), jnp.float32)],
    )(_sc_body)
    return fn(hm, muri, bounds)


def _unpack_perm():
    p = [0] * D
    for k in range(4):
        for i in range(16):
            p[32 * k + 2 * i] = 32 * k + i
            p[32 * k + 2 * i + 1] = 32 * k + 16 + i
    return p


@jax.jit
def kernel(x, is_center, unique_residue_index, not_pad_mask, W):
    del is_center
    uri = unique_residue_index.astype(jnp.int32)
    muri = jnp.where(not_pad_mask, uri, -1)

    hm = _matmul_relu(x, W)

    edges = jnp.arange(0, RPAD + RPT, RPT, dtype=jnp.int32)
    bounds = jnp.searchsorted(uri, edges).astype(jnp.int32)
    bounds = jnp.concatenate(
        [bounds, jnp.zeros((80 - NPART - 1,), jnp.int32)])

    out, cnt = _sc_segment_mean(hm, muri, bounds)
    node_emb = out[:R]
    residue_mask = cnt[:R, 0].astype(bool)
    return node_emb, residue_mask


# speedup vs baseline: 3.9716x; 1.0171x over previous
import functools

import jax
import jax.numpy as jnp
from jax import lax
from jax.experimental import pallas as pl
from jax.experimental.pallas import tpu as pltpu
from jax.experimental.pallas import tpu_sc as plsc

N = 320000
D = 128
R = 20000

NTILES = 32
NPART = 64
RPT = 320
RPAD = RPT * NPART
ROWS = 324
DUMP = 320
C = 128

_BL = 16000


def _mm_body(x_ref, w_ref, o_ref):
    h = jax.lax.dot_general(x_ref[...].astype(jnp.bfloat16),
                            w_ref[...].astype(jnp.bfloat16),
                            (((1,), (1,)), ((), ())),
                            preferred_element_type=jnp.float32)
    o_ref[...] = jnp.maximum(h, 0.0).astype(jnp.bfloat16)


def _matmul_relu(x, w):
    return pl.pallas_call(
        _mm_body,
        grid=(N // _BL,),
        in_specs=[pl.BlockSpec((_BL, D), lambda i: (i, 0)),
                  pl.BlockSpec((D, D), lambda i: (0, 0))],
        out_specs=pl.BlockSpec((_BL, D), lambda i: (i, 0)),
        out_shape=jax.ShapeDtypeStruct((N, D), jnp.bfloat16),
    )(x, w)


def _sc_body(hm_hbm, uri_hbm, bounds_hbm,
             out_hbm, cnt_hbm):
    pl.run_scoped(
        functools.partial(_sc_inner, hm_hbm, uri_hbm, bounds_hbm,
                          out_hbm, cnt_hbm),
        pltpu.VMEM((ROWS, D), jnp.float32),
        pltpu.VMEM((ROWS, 16), jnp.float32),
        pltpu.VMEM((C // 2, D), jnp.int32),
        pltpu.VMEM((C // 2, D), jnp.int32),
        pltpu.VMEM((C,), jnp.int32),
        pltpu.VMEM((C,), jnp.int32),
        pltpu.VMEM((80,), jnp.int32),
        pltpu.SemaphoreType.DMA,
        pltpu.SemaphoreType.DMA,
    )


def _sc_inner(hm_hbm, uri_hbm, bounds_hbm,
              out_hbm, cnt_hbm,
              accum, cnta, ch0, ch1, ix0, ix1, bvm, sem0, sem1):
    info = plsc.get_sparse_core_info()
    nc = info.num_cores
    sid = lax.axis_index("s")
    wid = sid * nc + lax.axis_index("c")

    zero16 = jnp.zeros((16,), jnp.float32)
    one16 = jnp.ones((16,), jnp.float32)
    himask = jnp.full((16,), -65536, jnp.int32)
    hm32_hbm = hm_hbm.bitcast(jnp.int32)
    pltpu.sync_copy(bounds_hbm, bvm)
    bv = bvm[pl.ds(2 * wid, 16)]

    bufs = ((ch0, ix0, sem0), (ch1, ix1, sem1))

    for h in range(2):
        b0 = bv[h]
        b1 = bv[h + 1]
        a0 = (b0 // 16) * 16
        nch = (b1 - a0 + (C - 1)) // C
        nch2 = ((nch + 1) // 2) * 2
        r0 = (2 * wid + h) * RPT

        def _zrow(r, carry):
            for j in range(8):
                accum[r, pl.ds(16 * j, 16)] = zero16
            cnta[r, :] = zero16
            return carry

        lax.fori_loop(0, ROWS, _zrow, 0)

        def _dstart(c, a0=a0):
            return jnp.minimum(a0 + c * C, N - C)

        def _issue(c, buf, a0=a0):
            ch, ix, sem = buf
            ds0 = _dstart(c, a0)
            row0 = pl.multiple_of(ds0 // 2, 8)
            pltpu.async_copy(hm32_hbm.at[pl.ds(row0, C // 2)], ch, sem)
            pltpu.async_copy(uri_hbm.at[pl.ds(ds0, C)], ix, sem)

        def _drain(buf):
            ch, ix, sem = buf
            pltpu.make_async_copy(
                hm32_hbm.at[pl.ds(0, C // 2)], ch, sem).wait()
            pltpu.make_async_copy(uri_hbm.at[pl.ds(0, C)], ix, sem).wait()

        def _process(c, buf, b0=b0, b1=b1, a0=a0, r0=r0):
            ch, ix, _ = buf
            start = a0 + c * C
            ds0 = _dstart(c, a0)

            def _group(g, carry2):
                pos = ds0 + g * 16 + lax.iota(jnp.int32, 16)
                iv = ix[pl.ds(g * 16, 16)]
                ok = (pos >= b0) & (pos >= start) & (pos < b1) & (iv >= 0)
                lv = jnp.where(ok, iv - r0, DUMP)
                for p in range(8):
                    re = lv[2 * p]
                    ro = lv[2 * p + 1]
                    plsc.addupdate(cnta.at[re, :], one16)
                    plsc.addupdate(cnta.at[ro, :], one16)
                    vrow = g * 8 + p
                    for k in range(8):
                        sl = pl.ds(16 * k, 16)
                        v = ch[vrow, sl]
                        fe = lax.bitcast_convert_type(v << 16, jnp.float32)
                        fo = lax.bitcast_convert_type(v & himask, jnp.float32)
                        plsc.addupdate(accum.at[re, sl], fe)
                        plsc.addupdate(accum.at[ro, sl], fo)
                return carry2

            lax.fori_loop(0, C // 16, _group, 0)

        _issue(0, bufs[0])

        def _pair(cc, carry):
            _issue(cc + 1, bufs[1])
            _drain(bufs[0])
            _process(cc, bufs[0])
            _issue(cc + 2, bufs[0])
            _drain(bufs[1])
            _process(cc + 1, bufs[1])
            return carry

        lax.fori_loop(0, nch2 // 2, lambda i, c: _pair(2 * i, c), 0)
        _drain(bufs[0])

        def _div(r, carry):
            cv = cnta[r, :]
            scale = 1.0 / jnp.maximum(cv, 1.0)
            for j in range(8):
                sl = pl.ds(16 * j, 16)
                accum[r, sl] = accum[r, sl] * scale
            cnta[r, :] = jnp.where(cv > 0.0, 1.0, 0.0)
            return carry

        lax.fori_loop(0, RPT, _div, 0)

        pltpu.sync_copy(accum.at[pl.ds(0, RPT)], out_hbm.at[pl.ds(r0, RPT)])
        pltpu.sync_copy(cnta.at[pl.ds(0, RPT)], cnt_hbm.at[pl.ds(r0, RPT)])


def _sc_segment_mean(hm, muri, bounds):
    mesh = plsc.VectorSubcoreMesh(core_axis_name="c", subcore_axis_name="s")
    fn = functools.partial(
        pl.kernel,
        mesh=mesh,
        out_type=[jax.ShapeDtypeStruct((RPAD, D), jnp.float32),
                  jax.ShapeDtypeStruct((RPAD, 16), jnp.float32)],
    )(_sc_body)
    return fn(hm, muri, bounds)


def _unpack_perm():
    p = [0] * D
    for k in range(4):
        for i in range(16):
            p[32 * k + 2 * i] = 32 * k + i
            p[32 * k + 2 * i + 1] = 32 * k + 16 + i
    return p


@jax.jit
def kernel(x, is_center, unique_residue_index, not_pad_mask, W):
    del is_center
    uri = unique_residue_index.astype(jnp.int32)
    muri = jnp.where(not_pad_mask, uri, -1)

    hm = _matmul_relu(x, W)

    edges = jnp.arange(0, RPAD + RPT, RPT, dtype=jnp.int32)
    bounds = jnp.searchsorted(uri, edges).astype(jnp.int32)
    bounds = jnp.concatenate(
        [bounds, jnp.zeros((80 - NPART - 1,), jnp.int32)])

    out, cnt = _sc_segment_mean(hm, muri, bounds)
    node_emb = out[:R]
    residue_mask = cnt[:R, 0].astype(bool)
    return node_emb, residue_mask


# speedup vs baseline: 4.0640x; 1.0233x over previous
import functools

import jax
import jax.numpy as jnp
from jax import lax
from jax.experimental import pallas as pl
from jax.experimental.pallas import tpu as pltpu
from jax.experimental.pallas import tpu_sc as plsc

N = 320000
D = 128
R = 20000

NTILES = 32
NPART = 64
RPT = 320
RPAD = RPT * NPART
ROWS = 324
DUMP = 320
C = 128

_BL = 32000


def _mm_body(x_ref, w_ref, o_ref):
    h = jax.lax.dot_general(x_ref[...].astype(jnp.bfloat16),
                            w_ref[...].astype(jnp.bfloat16),
                            (((1,), (1,)), ((), ())),
                            preferred_element_type=jnp.float32)
    o_ref[...] = jnp.maximum(h, 0.0).astype(jnp.bfloat16)


def _matmul_relu(x, w):
    return pl.pallas_call(
        _mm_body,
        grid=(N // _BL,),
        in_specs=[pl.BlockSpec((_BL, D), lambda i: (i, 0)),
                  pl.BlockSpec((D, D), lambda i: (0, 0))],
        out_specs=pl.BlockSpec((_BL, D), lambda i: (i, 0)),
        out_shape=jax.ShapeDtypeStruct((N, D), jnp.bfloat16),
    )(x, w)


def _sc_body(hm_hbm, uri_hbm, bounds_hbm,
             out_hbm, cnt_hbm):
    pl.run_scoped(
        functools.partial(_sc_inner, hm_hbm, uri_hbm, bounds_hbm,
                          out_hbm, cnt_hbm),
        pltpu.VMEM((ROWS, D), jnp.float32),
        pltpu.VMEM((ROWS, 16), jnp.float32),
        pltpu.VMEM((C // 2, D), jnp.int32),
        pltpu.VMEM((C // 2, D), jnp.int32),
        pltpu.VMEM((C,), jnp.int32),
        pltpu.VMEM((C,), jnp.int32),
        pltpu.VMEM((80,), jnp.int32),
        pltpu.SemaphoreType.DMA,
        pltpu.SemaphoreType.DMA,
    )


def _sc_inner(hm_hbm, uri_hbm, bounds_hbm,
              out_hbm, cnt_hbm,
              accum, cnta, ch0, ch1, ix0, ix1, bvm, sem0, sem1):
    info = plsc.get_sparse_core_info()
    nc = info.num_cores
    sid = lax.axis_index("s")
    wid = sid * nc + lax.axis_index("c")

    zero16 = jnp.zeros((16,), jnp.float32)
    one16 = jnp.ones((16,), jnp.float32)
    himask = jnp.full((16,), -65536, jnp.int32)
    hm32_hbm = hm_hbm.bitcast(jnp.int32)
    pltpu.sync_copy(bounds_hbm, bvm)
    bv = bvm[pl.ds(2 * wid, 16)]

    bufs = ((ch0, ix0, sem0), (ch1, ix1, sem1))

    for h in range(2):
        b0 = bv[h]
        b1 = bv[h + 1]
        a0 = (b0 // 16) * 16
        nch = (b1 - a0 + (C - 1)) // C
        nch2 = ((nch + 1) // 2) * 2
        r0 = (2 * wid + h) * RPT

        def _zrow(r, carry):
            for j in range(8):
                accum[r, pl.ds(16 * j, 16)] = zero16
            cnta[r, :] = zero16
            return carry

        lax.fori_loop(0, ROWS, _zrow, 0)

        def _dstart(c, a0=a0):
            return jnp.minimum(a0 + c * C, N - C)

        def _issue(c, buf, a0=a0):
            ch, ix, sem = buf
            ds0 = _dstart(c, a0)
            row0 = pl.multiple_of(ds0 // 2, 8)
            pltpu.async_copy(hm32_hbm.at[pl.ds(row0, C // 2)], ch, sem)
            pltpu.async_copy(uri_hbm.at[pl.ds(ds0, C)], ix, sem)

        def _drain(buf):
            ch, ix, sem = buf
            pltpu.make_async_copy(
                hm32_hbm.at[pl.ds(0, C // 2)], ch, sem).wait()
            pltpu.make_async_copy(uri_hbm.at[pl.ds(0, C)], ix, sem).wait()

        def _process(c, buf, b0=b0, b1=b1, a0=a0, r0=r0):
            ch, ix, _ = buf
            start = a0 + c * C
            ds0 = _dstart(c, a0)

            def _group(g, carry2):
                pos = ds0 + g * 16 + lax.iota(jnp.int32, 16)
                iv = ix[pl.ds(g * 16, 16)]
                ok = (pos >= b0) & (pos >= start) & (pos < b1) & (iv >= 0)
                lv = jnp.where(ok, iv - r0, DUMP)
                for p in range(8):
                    re = lv[2 * p]
                    ro = lv[2 * p + 1]
                    plsc.addupdate(cnta.at[re, :], one16)
                    plsc.addupdate(cnta.at[ro, :], one16)
                    vrow = g * 8 + p
                    for k in range(8):
                        sl = pl.ds(16 * k, 16)
                        v = ch[vrow, sl]
                        fe = lax.bitcast_convert_type(v << 16, jnp.float32)
                        fo = lax.bitcast_convert_type(v & himask, jnp.float32)
                        plsc.addupdate(accum.at[re, sl], fe)
                        plsc.addupdate(accum.at[ro, sl], fo)
                return carry2

            lax.fori_loop(0, C // 16, _group, 0)

        _issue(0, bufs[0])

        def _pair(cc, carry):
            _issue(cc + 1, bufs[1])
            _drain(bufs[0])
            _process(cc, bufs[0])
            _issue(cc + 2, bufs[0])
            _drain(bufs[1])
            _process(cc + 1, bufs[1])
            return carry

        lax.fori_loop(0, nch2 // 2, lambda i, c: _pair(2 * i, c), 0)
        _drain(bufs[0])

        def _div(r, carry):
            cv = cnta[r, :]
            scale = 1.0 / jnp.maximum(cv, 1.0)
            for j in range(8):
                sl = pl.ds(16 * j, 16)
                accum[r, sl] = accum[r, sl] * scale
            cnta[r, :] = jnp.where(cv > 0.0, 1.0, 0.0)
            return carry

        lax.fori_loop(0, RPT, _div, 0)

        p = 2 * wid + h
        @pl.when(p < NPART - 2)
        def _full_write():
            pltpu.sync_copy(accum.at[pl.ds(0, RPT)], out_hbm.at[pl.ds(r0, RPT)])

        @pl.when(p == NPART - 2)
        def _tail_write():
            pltpu.sync_copy(accum.at[pl.ds(0, R - (NPART - 2) * RPT)],
                            out_hbm.at[pl.ds((NPART - 2) * RPT,
                                             R - (NPART - 2) * RPT)])

        pltpu.sync_copy(cnta.at[pl.ds(0, RPT)], cnt_hbm.at[pl.ds(r0, RPT)])


def _sc_segment_mean(hm, muri, bounds):
    mesh = plsc.VectorSubcoreMesh(core_axis_name="c", subcore_axis_name="s")
    fn = functools.partial(
        pl.kernel,
        mesh=mesh,
        out_type=[jax.ShapeDtypeStruct((R, D), jnp.float32),
                  jax.ShapeDtypeStruct((RPAD, 16), jnp.float32)],
    )(_sc_body)
    return fn(hm, muri, bounds)


def _unpack_perm():
    p = [0] * D
    for k in range(4):
        for i in range(16):
            p[32 * k + 2 * i] = 32 * k + i
            p[32 * k + 2 * i + 1] = 32 * k + 16 + i
    return p


@jax.jit
def kernel(x, is_center, unique_residue_index, not_pad_mask, W):
    del is_center
    uri = unique_residue_index.astype(jnp.int32)
    muri = jnp.where(not_pad_mask, uri, -1)

    hm = _matmul_relu(x, W)

    edges = jnp.arange(0, RPAD + RPT, RPT, dtype=jnp.int32)
    bounds = jnp.searchsorted(uri, edges).astype(jnp.int32)
    bounds = jnp.concatenate(
        [bounds, jnp.zeros((80 - NPART - 1,), jnp.int32)])

    node_emb, cnt = _sc_segment_mean(hm, muri, bounds)
    residue_mask = cnt[:R, 0].astype(bool)
    return node_emb, residue_mask
